# Initial kernel scaffold; baseline (speedup 1.0000x reference)
import jax
import jax.numpy as jnp
from jax.experimental import pallas as pl


def kernel(latent, embedding_weight):
    raise NotImplementedError("write your pallas kernel here")


# speedup vs baseline: 4.6857x; 4.6857x over previous
import jax
import jax.numpy as jnp
from jax.experimental import pallas as pl
from jax.experimental.pallas import tpu as pltpu

_N_EMB = 1024
_DIM = 64
_B = 32
_RPB = 1024
_N_ROWS = _B * _RPB
_COMMIT = 0.25


def _vq_body(lat_ref, w_ref, dist_ref, enc_ref, qt_ref, cnt_ref, loss_ref,
             perp_ref):
    i = pl.program_id(0)
    lm = lat_ref[0]
    w = w_ref[...]
    mm = jax.lax.dot_general(lm, w, (((0,), (1,)), ((), ())),
                             preferred_element_type=jnp.float32)
    rowsq = jnp.sum(lm * lm, axis=0).reshape(_RPB, 1)
    wsq = jnp.sum(w * w, axis=1).reshape(1, _N_EMB)
    dist = (rowsq + wsq) - 2.0 * mm
    dist_ref[...] = dist

    minv = jnp.min(dist, axis=1, keepdims=True)
    cols = jax.lax.broadcasted_iota(jnp.int32, (_RPB, _N_EMB), 1)
    idx = jnp.min(jnp.where(dist == minv, cols, _N_EMB), axis=1, keepdims=True)
    enc = (cols == idx).astype(jnp.float32)
    enc_ref[...] = enc

    qt = jax.lax.dot_general(w, enc, (((0,), (1,)), ((), ())),
                             preferred_element_type=jnp.float32)
    qt_ref[0] = qt

    pcnt = jnp.sum(enc, axis=0).reshape(1, _N_EMB)
    ploss = jnp.sum((qt - lm) ** 2)

    @pl.when(i == 0)
    def _init():
        cnt_ref[...] = pcnt
        loss_ref[0, 0] = ploss

    @pl.when(i > 0)
    def _acc():
        cnt_ref[...] += pcnt
        loss_ref[0, 0] += ploss

    @pl.when(i == _B - 1)
    def _fin():
        avg = cnt_ref[...] * (1.0 / _N_ROWS)
        perp_ref[0, 0] = jnp.exp(-jnp.sum(avg * jnp.log(avg + 1e-10)))
        loss_ref[0, 0] = loss_ref[0, 0] * (_COMMIT / (_N_ROWS * _DIM))


_IN_SPECS = [
    pl.BlockSpec((1, _DIM, _RPB), lambda i: (i, 0, 0)),
    pl.BlockSpec((_N_EMB, _DIM), lambda i: (0, 0)),
]
_OUT_SPECS = [
    pl.BlockSpec((_RPB, _N_EMB), lambda i: (i, 0)),
    pl.BlockSpec((_RPB, _N_EMB), lambda i: (i, 0)),
    pl.BlockSpec((1, _DIM, _RPB), lambda i: (i, 0, 0)),
    pl.BlockSpec((1, _N_EMB), lambda i: (0, 0)),
    pl.BlockSpec((1, 1), lambda i: (0, 0), memory_space=pltpu.SMEM),
    pl.BlockSpec((1, 1), lambda i: (0, 0), memory_space=pltpu.SMEM),
]
_OUT_SHAPE = [
    jax.ShapeDtypeStruct((_N_ROWS, _N_EMB), jnp.float32),
    jax.ShapeDtypeStruct((_N_ROWS, _N_EMB), jnp.float32),
    jax.ShapeDtypeStruct((_B, _DIM, _RPB), jnp.float32),
    jax.ShapeDtypeStruct((1, _N_EMB), jnp.float32),
    jax.ShapeDtypeStruct((1, 1), jnp.float32),
    jax.ShapeDtypeStruct((1, 1), jnp.float32),
]


def kernel(latent, embedding_weight):
    lat3 = latent.reshape(_B, _DIM, _RPB)
    dist, enc, qt, _cnt, loss, perp = pl.pallas_call(
        _vq_body,
        grid=(_B,),
        in_specs=_IN_SPECS,
        out_specs=_OUT_SPECS,
        out_shape=_OUT_SHAPE,
        compiler_params=pltpu.CompilerParams(
            dimension_semantics=("arbitrary",)),
    )(lat3, embedding_weight)
    quantized_out = qt.reshape(_B, _DIM, 32, 32)
    return quantized_out, loss[0, 0], perp[0, 0], enc, dist
